# baseline (speedup 1.0000x reference)
import jax
import jax.numpy as jnp
from jax.experimental import pallas as pl

_BS = 2048


def _fused_kernel(emb_ref, mask_ref, wt_ref, out_ref):
    x = emb_ref[0].astype(jnp.bfloat16)
    mm = jnp.dot(x, wt_ref[...], preferred_element_type=jnp.float32)
    m = mask_ref[0] > 0
    out_ref[0] = jnp.where(m, mm, -jnp.inf)


def kernel(emb_sentences, att_sentences, W):
    B, S, D = emb_sentences.shape
    L = W.shape[0]
    mask = att_sentences[:, :, None].astype(jnp.float32)
    wt = W.T.astype(jnp.bfloat16)

    grid = (B, S // _BS)
    out = pl.pallas_call(
        _fused_kernel,
        grid=grid,
        in_specs=[
            pl.BlockSpec((1, _BS, D), lambda b, i: (b, i, 0)),
            pl.BlockSpec((1, _BS, 1), lambda b, i: (b, i, 0)),
            pl.BlockSpec((D, L), lambda b, i: (0, 0)),
        ],
        out_specs=pl.BlockSpec((1, _BS, L), lambda b, i: (b, i, 0)),
        out_shape=jax.ShapeDtypeStruct((B, S, L), jnp.float32),
    )(emb_sentences, mask, wt)
    return out


# speedup vs baseline: 1.6337x; 1.6337x over previous
import jax
import jax.numpy as jnp
from jax import lax
from jax.experimental import pallas as pl

_BS = 2048


def _fused_kernel(emb_ref, mask_ref, w_ref, out_ref):
    x = emb_ref[0].astype(jnp.bfloat16)
    wb = w_ref[...].astype(jnp.bfloat16)
    mm = lax.dot_general(wb, x, (((1,), (1,)), ((), ())),
                         preferred_element_type=jnp.float32)
    m = mask_ref[0] > 0
    out_ref[0] = jnp.where(m, mm, -jnp.inf)


def kernel(emb_sentences, att_sentences, W):
    B, S, D = emb_sentences.shape
    L = W.shape[0]
    mask = att_sentences[:, None, :].astype(jnp.float32)

    grid = (B, S // _BS)
    out_t = pl.pallas_call(
        _fused_kernel,
        grid=grid,
        in_specs=[
            pl.BlockSpec((1, _BS, D), lambda b, i: (b, i, 0)),
            pl.BlockSpec((1, 1, _BS), lambda b, i: (b, 0, i)),
            pl.BlockSpec((L, D), lambda b, i: (0, 0)),
        ],
        out_specs=pl.BlockSpec((1, L, _BS), lambda b, i: (b, 0, i)),
        out_shape=jax.ShapeDtypeStruct((B, L, S), jnp.float32),
    )(emb_sentences, mask, W)
    return jnp.swapaxes(out_t, 1, 2)
